# speedup vs baseline: 1.2462x; 1.2462x over previous
import functools

import jax
import jax.numpy as jnp
from jax import lax
from jax.experimental import pallas as pl
from jax.experimental.pallas import tpu as pltpu
from jax.experimental.pallas import tpu_sc as plsc

_LANES = 16


@functools.lru_cache(maxsize=None)
def _build(n_tok, vocab, hidden, max_pos):
    info = plsc.get_sparse_core_info()
    num_workers = info.num_cores * info.num_subcores
    assert n_tok % num_workers == 0
    tokens_per_worker = n_tok // num_workers
    chunk = 32
    assert tokens_per_worker % chunk == 0
    n_chunks = tokens_per_worker // chunk
    n_vec = hidden // _LANES

    mesh = plsc.VectorSubcoreMesh(core_axis_name="c", subcore_axis_name="s")

    @functools.partial(
        pl.kernel,
        mesh=mesh,
        out_type=jax.ShapeDtypeStruct((n_tok, hidden), jnp.float32),
        scratch_types=[
            pltpu.VMEM((chunk,), jnp.int32),
            pltpu.VMEM((chunk,), jnp.int32),
            pltpu.VMEM((chunk, hidden), jnp.float32),
            pltpu.VMEM((chunk, hidden), jnp.float32),
            pltpu.SemaphoreType.DMA,
            pltpu.SemaphoreType.DMA,
        ],
    )
    def emb_kernel(ids_hbm, pids_hbm, word_hbm, pos_hbm, out_hbm,
                   idw_v, idp_v, wbuf, pbuf, sem_w, sem_p):
        wid = lax.axis_index("s") * info.num_cores + lax.axis_index("c")
        base = wid * tokens_per_worker

        def chunk_body(c, carry):
            tok = pl.multiple_of(base + c * chunk, chunk)
            pltpu.sync_copy(ids_hbm.at[pl.ds(tok, chunk)], idw_v)
            pltpu.sync_copy(pids_hbm.at[pl.ds(tok, chunk)], idp_v)
            cp_w = pltpu.async_copy(word_hbm.at[idw_v], wbuf, sem_w)
            cp_p = pltpu.async_copy(pos_hbm.at[idp_v], pbuf, sem_p)
            cp_w.wait()
            cp_p.wait()

            def add_row(t, acc):
                for j in range(n_vec):
                    sl = pl.ds(j * _LANES, _LANES)
                    plsc.addupdate(wbuf.at[t, sl], pbuf[t, sl])
                return acc

            lax.fori_loop(0, chunk, add_row, 0)
            pltpu.sync_copy(wbuf, out_hbm.at[pl.ds(tok, chunk)])
            return carry

        lax.fori_loop(0, n_chunks, chunk_body, 0)

    return emb_kernel


def kernel(input_ids, position_ids, word_embeddings_weight, position_embeddings_weight):
    b, s = input_ids.shape
    vocab, hidden = word_embeddings_weight.shape
    max_pos = position_embeddings_weight.shape[0]
    fn = _build(b * s, vocab, hidden, max_pos)
    out = fn(
        input_ids.reshape(-1),
        position_ids.reshape(-1),
        word_embeddings_weight,
        position_embeddings_weight,
    )
    return out.reshape(b, s, hidden)


# speedup vs baseline: 1.3519x; 1.0848x over previous
import functools

import jax
import jax.numpy as jnp
from jax import lax
from jax.experimental import pallas as pl
from jax.experimental.pallas import tpu as pltpu
from jax.experimental.pallas import tpu_sc as plsc

_LANES = 16


@functools.lru_cache(maxsize=None)
def _build(n_tok, vocab, hidden, max_pos):
    info = plsc.get_sparse_core_info()
    num_workers = info.num_cores * info.num_subcores
    assert n_tok % num_workers == 0
    tokens_per_worker = n_tok // num_workers
    chunk = 16
    assert tokens_per_worker % (2 * chunk) == 0
    n_chunks = tokens_per_worker // chunk
    n_vec = hidden // _LANES

    mesh = plsc.VectorSubcoreMesh(core_axis_name="c", subcore_axis_name="s")

    @functools.partial(
        pl.kernel,
        mesh=mesh,
        out_type=jax.ShapeDtypeStruct((n_tok, hidden), jnp.float32),
        scratch_types=[
            pltpu.VMEM((tokens_per_worker,), jnp.int32),
            pltpu.VMEM((tokens_per_worker,), jnp.int32),
            pltpu.VMEM((chunk, hidden), jnp.float32),
            pltpu.VMEM((chunk, hidden), jnp.float32),
            pltpu.VMEM((chunk, hidden), jnp.float32),
            pltpu.VMEM((chunk, hidden), jnp.float32),
            pltpu.SemaphoreType.DMA,
            pltpu.SemaphoreType.DMA,
            pltpu.SemaphoreType.DMA,
            pltpu.SemaphoreType.DMA,
        ],
    )
    def emb_kernel(ids_hbm, pids_hbm, word_hbm, pos_hbm, out_hbm,
                   idw, idp, wbuf0, wbuf1, pbuf0, pbuf1,
                   sw0, sw1, sp0, sp1):
        wid = lax.axis_index("s") * info.num_cores + lax.axis_index("c")
        base = wid * tokens_per_worker
        pltpu.sync_copy(ids_hbm.at[pl.ds(base, tokens_per_worker)], idw)
        pltpu.sync_copy(pids_hbm.at[pl.ds(base, tokens_per_worker)], idp)

        wb = (wbuf0, wbuf1)
        pb = (pbuf0, pbuf1)
        sw = (sw0, sw1)
        sp = (sp0, sp1)

        def issue(c, b):
            off = pl.multiple_of(c * chunk, chunk)
            pltpu.async_copy(word_hbm.at[idw.at[pl.ds(off, chunk)]], wb[b], sw[b])
            pltpu.async_copy(pos_hbm.at[idp.at[pl.ds(off, chunk)]], pb[b], sp[b])

        def wait(b):
            pltpu.make_async_copy(
                word_hbm.at[idw.at[pl.ds(0, chunk)]], wb[b], sw[b]).wait()
            pltpu.make_async_copy(
                pos_hbm.at[idp.at[pl.ds(0, chunk)]], pb[b], sp[b]).wait()

        def add_rows(b):
            wbuf, pbuf = wb[b], pb[b]

            def row(t, acc):
                for j in range(n_vec):
                    sl = pl.ds(j * _LANES, _LANES)
                    plsc.addupdate(wbuf.at[t, sl], pbuf[t, sl])
                return acc

            lax.fori_loop(0, chunk, row, 0)

        def step(c, b):
            wait(b)
            add_rows(b)
            out_off = pl.multiple_of(base + c * chunk, chunk)
            pltpu.sync_copy(wb[b], out_hbm.at[pl.ds(out_off, chunk)])

            @pl.when(c + 2 < n_chunks)
            def _():
                issue(c + 2, b)

        issue(0, 0)
        issue(1, 1)

        def body(cc, carry):
            c0 = cc * 2
            step(c0, 0)
            step(c0 + 1, 1)
            return carry

        lax.fori_loop(0, n_chunks // 2, body, 0)

    return emb_kernel


def kernel(input_ids, position_ids, word_embeddings_weight, position_embeddings_weight):
    b, s = input_ids.shape
    vocab, hidden = word_embeddings_weight.shape
    max_pos = position_embeddings_weight.shape[0]
    fn = _build(b * s, vocab, hidden, max_pos)
    out = fn(
        input_ids.reshape(-1),
        position_ids.reshape(-1),
        word_embeddings_weight,
        position_embeddings_weight,
    )
    return out.reshape(b, s, hidden)


# speedup vs baseline: 2.2579x; 1.6702x over previous
import functools

import jax
import jax.numpy as jnp
from jax import lax
from jax.experimental import pallas as pl
from jax.experimental.pallas import tpu as pltpu
from jax.experimental.pallas import tpu_sc as plsc

_LANES = 16


@functools.lru_cache(maxsize=None)
def _build(n_tok, vocab, hidden, max_pos):
    info = plsc.get_sparse_core_info()
    num_workers = info.num_cores * info.num_subcores
    assert n_tok % num_workers == 0
    tokens_per_worker = n_tok // num_workers
    chunk = 16
    assert tokens_per_worker % (2 * chunk) == 0
    n_chunks = tokens_per_worker // chunk
    n_vec = hidden // _LANES

    mesh = plsc.VectorSubcoreMesh(core_axis_name="c", subcore_axis_name="s")

    @functools.partial(
        pl.kernel,
        mesh=mesh,
        out_type=jax.ShapeDtypeStruct((n_tok, hidden), jnp.float32),
        scratch_types=[
            pltpu.VMEM((tokens_per_worker,), jnp.int32),
            pltpu.VMEM((tokens_per_worker,), jnp.int32),
            pltpu.VMEM((chunk, hidden), jnp.float32),
            pltpu.VMEM((chunk, hidden), jnp.float32),
            pltpu.VMEM((chunk, hidden), jnp.float32),
            pltpu.VMEM((chunk, hidden), jnp.float32),
            pltpu.VMEM((chunk, hidden), jnp.float32),
            pltpu.VMEM((chunk, hidden), jnp.float32),
            pltpu.SemaphoreType.DMA,
            pltpu.SemaphoreType.DMA,
            pltpu.SemaphoreType.DMA,
            pltpu.SemaphoreType.DMA,
            pltpu.SemaphoreType.DMA,
            pltpu.SemaphoreType.DMA,
        ],
    )
    def emb_kernel(ids_hbm, pids_hbm, word_hbm, pos_hbm, out_hbm,
                   idw, idp, wbuf0, wbuf1, pbuf0, pbuf1, obuf0, obuf1,
                   sw0, sw1, sp0, sp1, so0, so1):
        wid = lax.axis_index("s") * info.num_cores + lax.axis_index("c")
        base = wid * tokens_per_worker
        pltpu.sync_copy(ids_hbm.at[pl.ds(base, tokens_per_worker)], idw)
        pltpu.sync_copy(pids_hbm.at[pl.ds(base, tokens_per_worker)], idp)

        wb = (wbuf0, wbuf1)
        pb = (pbuf0, pbuf1)
        ob = (obuf0, obuf1)
        sw = (sw0, sw1)
        sp = (sp0, sp1)
        so = (so0, so1)

        def issue(c, b):
            off = pl.multiple_of(c * chunk, chunk)
            pltpu.async_copy(word_hbm.at[idw.at[pl.ds(off, chunk)]], wb[b], sw[b])
            pltpu.async_copy(pos_hbm.at[idp.at[pl.ds(off, chunk)]], pb[b], sp[b])

        def wait_gathers(b):
            pltpu.make_async_copy(
                word_hbm.at[idw.at[pl.ds(0, chunk)]], wb[b], sw[b]).wait()
            pltpu.make_async_copy(
                pos_hbm.at[idp.at[pl.ds(0, chunk)]], pb[b], sp[b]).wait()

        def wait_scatter(b):
            pltpu.make_async_copy(
                ob[b], out_hbm.at[pl.ds(base, chunk)], so[b]).wait()

        def add_rows(b):
            wbuf, pbuf, obuf = wb[b], pb[b], ob[b]

            def row(t, acc):
                for j in range(n_vec):
                    sl = pl.ds(j * _LANES, _LANES)
                    obuf[t, sl] = wbuf[t, sl] + pbuf[t, sl]
                return acc

            lax.fori_loop(0, chunk, row, 0)

        def step(c, b):
            wait_gathers(b)

            @pl.when(c >= 2)
            def _():
                wait_scatter(b)

            add_rows(b)

            @pl.when(c + 2 < n_chunks)
            def _():
                issue(c + 2, b)

            out_off = pl.multiple_of(base + c * chunk, chunk)
            pltpu.async_copy(ob[b], out_hbm.at[pl.ds(out_off, chunk)], so[b])

        issue(0, 0)
        issue(1, 1)

        def body(cc, carry):
            c0 = cc * 2
            step(c0, 0)
            step(c0 + 1, 1)
            return carry

        lax.fori_loop(0, n_chunks // 2, body, 0)
        wait_scatter(0)
        wait_scatter(1)

    return emb_kernel


def kernel(input_ids, position_ids, word_embeddings_weight, position_embeddings_weight):
    b, s = input_ids.shape
    vocab, hidden = word_embeddings_weight.shape
    max_pos = position_embeddings_weight.shape[0]
    fn = _build(b * s, vocab, hidden, max_pos)
    out = fn(
        input_ids.reshape(-1),
        position_ids.reshape(-1),
        word_embeddings_weight,
        position_embeddings_weight,
    )
    return out.reshape(b, s, hidden)
